# baseline (speedup 1.0000x reference)
import functools

import jax
import jax.numpy as jnp
from jax import lax
from jax.experimental import pallas as pl
from jax.experimental.pallas import tpu as pltpu
from jax.experimental.pallas import tpu_sc as plsc

EPS = 1e-09
N = 10000
E = 640000
NW = 32
CH = 64
EP = 655360
NACC = 10112
NBUF = 5
LOOK = 3
SCH = 64
BE = 2560
BM = 1024
BN = 1280


def _sc_gather(table, idx):
    nch = idx.shape[1]
    ep = NW * nch * CH
    d = table.shape[1]
    mesh = plsc.VectorSubcoreMesh(core_axis_name="c", subcore_axis_name="s")

    def out_at(ref, j):
        if d == 128:
            return ref.at[pl.ds(j * CH, CH)]
        return ref.at[pl.ds(j * CH, CH), pl.ds(0, d)]

    @functools.partial(
        pl.kernel,
        out_type=jax.ShapeDtypeStruct((ep, 128), jnp.float32),
        mesh=mesh,
        compiler_params=pltpu.CompilerParams(use_tc_tiling_on_sc=False),
        scratch_types=[
            pltpu.VMEM((nch, CH), jnp.int32),
            pltpu.VMEM((NBUF, CH, d), jnp.float32),
            pltpu.SemaphoreType.DMA((NBUF,)),
            pltpu.SemaphoreType.DMA((NBUF,)),
        ],
    )
    def k(table_hbm, idx_hbm, out_hbm, idx_v, bufs_v, gsem, ssem):
        wid = lax.axis_index("s") * 2 + lax.axis_index("c")
        row0 = wid * nch
        pltpu.sync_copy(idx_hbm.at[wid], idx_v)

        for j in range(LOOK):
            pltpu.async_copy(table_hbm.at[idx_v.at[j]], bufs_v.at[j], gsem.at[j])

        def group(g, carry):
            for b in range(NBUF):
                j = g * NBUF + b
                pltpu.make_async_copy(
                    table_hbm.at[idx_v.at[0]], bufs_v.at[b], gsem.at[b]).wait()
                pltpu.async_copy(
                    bufs_v.at[b], out_at(out_hbm, row0 + j), ssem.at[b])
                jn = j + LOOK
                bn = (b + LOOK) % NBUF

                @pl.when(jn < nch)
                def _():
                    @pl.when(jn >= NBUF)
                    def _():
                        pltpu.make_async_copy(
                            bufs_v.at[bn], out_at(out_hbm, 0),
                            ssem.at[bn]).wait()
                    pltpu.async_copy(
                        table_hbm.at[idx_v.at[jn]], bufs_v.at[bn], gsem.at[bn])
            return carry

        lax.fori_loop(0, nch // NBUF, group, 0)
        for b in range(NBUF):
            pltpu.make_async_copy(
                bufs_v.at[b], out_at(out_hbm, 0), ssem.at[b]).wait()

    return k(table, idx)


def _sc_scatter_add(vals, idx, zeros, nbuf, look):
    nch = idx.shape[1]
    d = zeros.shape[1]
    rpt = NACC // 16
    mesh = plsc.VectorSubcoreMesh(core_axis_name="c", subcore_axis_name="s")

    scratch = [
        pltpu.VMEM((nch, SCH), jnp.int32),
        pltpu.VMEM((nbuf, SCH, 128), jnp.float32),
    ]
    if d < 128:
        scratch.append(pltpu.VMEM((nbuf, SCH, d), jnp.float32))
    scratch += [
        pltpu.VMEM_SHARED((NACC, d), jnp.float32),
        pltpu.SemaphoreType.DMA((nbuf,)),
        pltpu.SemaphoreType.DMA((nbuf,)),
    ]

    @functools.partial(
        pl.kernel,
        out_type=jax.ShapeDtypeStruct((2, NACC, d), jnp.float32),
        mesh=mesh,
        compiler_params=pltpu.CompilerParams(use_tc_tiling_on_sc=False),
        scratch_types=scratch,
    )
    def k(vals_hbm, idx_hbm, zeros_hbm, out_hbm, idx_v, bufs_v, *rest):
        if d < 128:
            cbufs_v, acc_sh, gsem, ssem = rest
        else:
            acc_sh, gsem, ssem = rest
            cbufs_v = None
        cid = lax.axis_index("c")
        sid = lax.axis_index("s")
        wid = sid * 2 + cid
        row0 = wid * nch
        pltpu.sync_copy(zeros_hbm.at[pl.ds(sid * rpt, rpt)],
                        acc_sh.at[pl.ds(sid * rpt, rpt)])
        plsc.subcore_barrier()
        pltpu.sync_copy(idx_hbm.at[wid], idx_v)

        def src(b):
            return bufs_v.at[b] if d == 128 else cbufs_v.at[b]

        def repack(b):
            if d == 128:
                return
            unroll = 8

            def rows(r0, carry):
                for u in range(unroll):
                    r = r0 * unroll + u
                    for q in range(d // 16):
                        cbufs_v[b, r, pl.ds(q * 16, 16)] = (
                            bufs_v[b, r, pl.ds(q * 16, 16)])
                return carry

            lax.fori_loop(0, SCH // unroll, rows, 0)

        for j in range(look):
            pltpu.async_copy(
                vals_hbm.at[pl.ds((row0 + j) * SCH, SCH)], bufs_v.at[j],
                gsem.at[j])

        def group(g, carry):
            for b in range(nbuf):
                j = g * nbuf + b
                pltpu.make_async_copy(
                    vals_hbm.at[pl.ds(0, SCH)], bufs_v.at[b], gsem.at[b]).wait()
                repack(b)
                pltpu.async_copy(
                    src(b), acc_sh.at[idx_v.at[j]], ssem.at[b], add=True)
                jn = j + look
                bn = (b + look) % nbuf

                @pl.when(jn < nch)
                def _():
                    @pl.when(jn >= nbuf)
                    def _():
                        pltpu.make_async_copy(
                            src(bn), acc_sh.at[idx_v.at[0]],
                            ssem.at[bn]).wait()
                    pltpu.async_copy(
                        vals_hbm.at[pl.ds((row0 + jn) * SCH, SCH)], bufs_v.at[bn],
                        gsem.at[bn])
            return carry

        lax.fori_loop(0, nch // nbuf, group, 0)
        for b in range(nbuf):
            pltpu.make_async_copy(
                src(b), acc_sh.at[idx_v.at[0]], ssem.at[b]).wait()
        plsc.subcore_barrier()
        pltpu.sync_copy(acc_sh.at[pl.ds(sid * rpt, rpt)],
                        out_hbm.at[cid].at[pl.ds(sid * rpt, rpt)])

    return k(vals, idx, zeros)


def _make_edge(dg):
    def body(g_ref, ea_ref, w_ref, b_ref, nw_ref, o_ref):
        e = jnp.dot(ea_ref[...], w_ref[...], preferred_element_type=jnp.float32)
        m = jax.nn.relu(g_ref[:, :dg] + e + b_ref[...])
        o_ref[...] = jnp.dot(m, nw_ref[...], preferred_element_type=jnp.float32)

    def edge(g, ea, w, b, nw):
        return pl.pallas_call(
            body,
            grid=(EP // BE,),
            in_specs=[
                pl.BlockSpec((BE, 128), lambda i: (i, 0)),
                pl.BlockSpec((BE, 16), lambda i: (jnp.minimum(i, E // BE - 1), 0)),
                pl.BlockSpec((16, dg), lambda i: (0, 0)),
                pl.BlockSpec((1, dg), lambda i: (0, 0)),
                pl.BlockSpec((dg, 128), lambda i: (0, 0)),
            ],
            out_specs=pl.BlockSpec((BE, 128), lambda i: (i, 0)),
            out_shape=jax.ShapeDtypeStruct((EP, 128), jnp.float32),
        )(g, ea, w, b, nw)

    return edge


_edge1 = _make_edge(128)
_edge2 = _make_edge(64)


def _gram_body(a_ref, b_ref, o_ref):
    o_ref[...] = jax.lax.dot_general(
        a_ref[...], b_ref[...], (((1,), (1,)), ((), ())),
        preferred_element_type=jnp.float32)


def _gram(h):
    n = h.shape[0]
    grid = (pl.cdiv(n, BM), pl.cdiv(n, BN))
    return pl.pallas_call(
        _gram_body,
        grid=grid,
        in_specs=[
            pl.BlockSpec((BM, h.shape[1]), lambda i, j: (i, 0)),
            pl.BlockSpec((BN, h.shape[1]), lambda i, j: (j, 0)),
        ],
        out_specs=pl.BlockSpec((BM, BN), lambda i, j: (i, j)),
        out_shape=jax.ShapeDtypeStruct((n, n), jnp.float32),
    )(h, h)


def _pad128(w):
    return jnp.pad(w, ((0, 0), (0, 128 - w.shape[1])))


def kernel(x, edge_index, edge_attr, emb, lin_edge1_w, lin_edge1_b, nn1_w, nn1_b,
           lin_edge2_w, lin_edge2_b, nn2_w, nn2_b):
    src = jnp.concatenate([edge_index[0], jnp.zeros((EP - E,), jnp.int32)])
    dst = jnp.concatenate(
        [edge_index[1], jnp.full((EP - E,), N, jnp.int32)])
    src3 = src.reshape(NW, EP // (NW * CH), CH)
    dst3 = dst.reshape(NW, EP // (NW * SCH), SCH)
    hx = jnp.take(emb, x[:, 0], axis=0)

    hs = _sc_gather(hx, src3)
    p1 = _edge1(hs, edge_attr, lin_edge1_w, lin_edge1_b.reshape(1, 128),
                _pad128(nn1_w))
    part = _sc_scatter_add(p1, dst3, jnp.zeros((NACC, 128), jnp.float32),
                           nbuf=2, look=1)
    aggr = (part[0] + part[1])[:N, :64]
    h1 = jax.nn.leaky_relu(aggr + (1.0 + EPS) * (hx @ nn1_w) + nn1_b,
                           negative_slope=0.01)

    g = _sc_gather(h1, src3)
    p2 = _edge2(g, edge_attr, lin_edge2_w, lin_edge2_b.reshape(1, 64),
                _pad128(nn2_w))
    part = _sc_scatter_add(p2, dst3, jnp.zeros((NACC, 32), jnp.float32),
                           nbuf=4, look=2)
    aggr = (part[0] + part[1])[:N]
    h2 = aggr + (1.0 + EPS) * (h1 @ nn2_w) + nn2_b

    return _gram(h2)


# speedup vs baseline: 1.3832x; 1.3832x over previous
import functools

import jax
import jax.numpy as jnp
from jax import lax
from jax.experimental import pallas as pl
from jax.experimental.pallas import tpu as pltpu
from jax.experimental.pallas import tpu_sc as plsc

EPS = 1e-09
N = 10000
E = 640000
NW = 32
CH = 64
EP = 655360
NACC = 10112
NBUF = 5
LOOK = 3
SCH = 64
BE = 5120
BM = 1024
BN = 1280


def _sc_class_gather(xflat, idx):
    nch = idx.shape[1]
    per_w = nch * CH
    ep = NW * per_w
    mesh = plsc.VectorSubcoreMesh(core_axis_name="c", subcore_axis_name="s")

    @functools.partial(
        pl.kernel,
        out_type=jax.ShapeDtypeStruct((ep,), jnp.int32),
        mesh=mesh,
        compiler_params=pltpu.CompilerParams(use_tc_tiling_on_sc=False,
                                             needs_layout_passes=False),
        scratch_types=[
            pltpu.VMEM((N,), jnp.int32),
            pltpu.VMEM((per_w,), jnp.int32),
            pltpu.VMEM((per_w,), jnp.int32),
        ],
    )
    def k(x_hbm, idx_hbm, out_hbm, x_v, idx_v, c_v):
        wid = lax.axis_index("s") * 2 + lax.axis_index("c")
        pltpu.sync_copy(x_hbm, x_v)
        pltpu.sync_copy(idx_hbm.at[wid], idx_v)

        def grp(t, carry):
            s16 = idx_v[pl.ds(t * 16, 16)]
            c_v[pl.ds(t * 16, 16)] = plsc.load_gather(x_v, [s16])
            return carry

        lax.fori_loop(0, per_w // 16, grp, 0)
        pltpu.sync_copy(c_v, out_hbm.at[pl.ds(wid * per_w, per_w)])

    return k(xflat, idx.reshape(NW, per_w))


def _sc_gather(table, idx):
    nch = idx.shape[1]
    ep = NW * nch * CH
    d = table.shape[1]
    mesh = plsc.VectorSubcoreMesh(core_axis_name="c", subcore_axis_name="s")

    def out_at(ref, j):
        return ref.at[pl.ds(j * CH, CH)]

    @functools.partial(
        pl.kernel,
        out_type=jax.ShapeDtypeStruct((ep, d), jnp.float32),
        mesh=mesh,
        compiler_params=pltpu.CompilerParams(use_tc_tiling_on_sc=False),
        scratch_types=[
            pltpu.VMEM((nch, CH), jnp.int32),
            pltpu.VMEM((NBUF, CH, d), jnp.float32),
            pltpu.SemaphoreType.DMA((NBUF,)),
            pltpu.SemaphoreType.DMA((NBUF,)),
        ],
    )
    def k(table_hbm, idx_hbm, out_hbm, idx_v, bufs_v, gsem, ssem):
        wid = lax.axis_index("s") * 2 + lax.axis_index("c")
        row0 = wid * nch
        pltpu.sync_copy(idx_hbm.at[wid], idx_v)

        for j in range(LOOK):
            pltpu.async_copy(table_hbm.at[idx_v.at[j]], bufs_v.at[j], gsem.at[j])

        def group(g, carry):
            for b in range(NBUF):
                j = g * NBUF + b
                pltpu.make_async_copy(
                    table_hbm.at[idx_v.at[0]], bufs_v.at[b], gsem.at[b]).wait()
                pltpu.async_copy(
                    bufs_v.at[b], out_at(out_hbm, row0 + j), ssem.at[b])
                jn = j + LOOK
                bn = (b + LOOK) % NBUF

                @pl.when(jn < nch)
                def _():
                    @pl.when(jn >= NBUF)
                    def _():
                        pltpu.make_async_copy(
                            bufs_v.at[bn], out_at(out_hbm, 0),
                            ssem.at[bn]).wait()
                    pltpu.async_copy(
                        table_hbm.at[idx_v.at[jn]], bufs_v.at[bn], gsem.at[bn])
            return carry

        lax.fori_loop(0, nch // NBUF, group, 0)
        for b in range(NBUF):
            pltpu.make_async_copy(
                bufs_v.at[b], out_at(out_hbm, 0), ssem.at[b]).wait()

    return k(table, idx)


def _sc_scatter_add(vals, idx, zeros, nbuf, look):
    nch = idx.shape[1]
    d = zeros.shape[1]
    rpt = NACC // 16
    mesh = plsc.VectorSubcoreMesh(core_axis_name="c", subcore_axis_name="s")

    scratch = [
        pltpu.VMEM((nch, SCH), jnp.int32),
        pltpu.VMEM((nbuf, SCH, 128), jnp.float32),
    ]
    if d < 128:
        scratch.append(pltpu.VMEM((nbuf, SCH, d), jnp.float32))
    scratch += [
        pltpu.VMEM_SHARED((NACC, d), jnp.float32),
        pltpu.SemaphoreType.DMA((nbuf,)),
        pltpu.SemaphoreType.DMA((nbuf,)),
    ]

    @functools.partial(
        pl.kernel,
        out_type=jax.ShapeDtypeStruct((2, NACC, d), jnp.float32),
        mesh=mesh,
        compiler_params=pltpu.CompilerParams(use_tc_tiling_on_sc=False),
        scratch_types=scratch,
    )
    def k(vals_hbm, idx_hbm, zeros_hbm, out_hbm, idx_v, bufs_v, *rest):
        if d < 128:
            cbufs_v, acc_sh, gsem, ssem = rest
        else:
            acc_sh, gsem, ssem = rest
            cbufs_v = None
        cid = lax.axis_index("c")
        sid = lax.axis_index("s")
        wid = sid * 2 + cid
        row0 = wid * nch
        pltpu.sync_copy(zeros_hbm.at[pl.ds(sid * rpt, rpt)],
                        acc_sh.at[pl.ds(sid * rpt, rpt)])
        plsc.subcore_barrier()
        pltpu.sync_copy(idx_hbm.at[wid], idx_v)

        def src(b):
            return bufs_v.at[b] if d == 128 else cbufs_v.at[b]

        def repack(b):
            if d == 128:
                return
            unroll = 8

            def rows(r0, carry):
                for u in range(unroll):
                    r = r0 * unroll + u
                    for q in range(d // 16):
                        cbufs_v[b, r, pl.ds(q * 16, 16)] = (
                            bufs_v[b, r, pl.ds(q * 16, 16)])
                return carry

            lax.fori_loop(0, SCH // unroll, rows, 0)

        for j in range(look):
            pltpu.async_copy(
                vals_hbm.at[pl.ds((row0 + j) * SCH, SCH)], bufs_v.at[j],
                gsem.at[j])

        def group(g, carry):
            for b in range(nbuf):
                j = g * nbuf + b
                pltpu.make_async_copy(
                    vals_hbm.at[pl.ds(0, SCH)], bufs_v.at[b], gsem.at[b]).wait()
                repack(b)
                pltpu.async_copy(
                    src(b), acc_sh.at[idx_v.at[j]], ssem.at[b], add=True)
                jn = j + look
                bn = (b + look) % nbuf

                @pl.when(jn < nch)
                def _():
                    @pl.when(jn >= nbuf)
                    def _():
                        pltpu.make_async_copy(
                            src(bn), acc_sh.at[idx_v.at[0]],
                            ssem.at[bn]).wait()
                    pltpu.async_copy(
                        vals_hbm.at[pl.ds((row0 + jn) * SCH, SCH)], bufs_v.at[bn],
                        gsem.at[bn])
            return carry

        lax.fori_loop(0, nch // nbuf, group, 0)
        for b in range(nbuf):
            pltpu.make_async_copy(
                src(b), acc_sh.at[idx_v.at[0]], ssem.at[b]).wait()
        plsc.subcore_barrier()
        pltpu.sync_copy(acc_sh.at[pl.ds(sid * rpt, rpt)],
                        out_hbm.at[cid].at[pl.ds(sid * rpt, rpt)])

    return k(vals, idx, zeros)


def _edge1_body(c_ref, ea_ref, emb_ref, w_ref, b_ref, nw_ref, o_ref):
    c = c_ref[...]
    oht = (jax.lax.broadcasted_iota(jnp.int32, (20, BE), 0)
           == c[None, :]).astype(jnp.float32)
    hsrc = jax.lax.dot_general(
        oht, emb_ref[...], (((0,), (0,)), ((), ())),
        preferred_element_type=jnp.float32)
    e = jnp.dot(ea_ref[...], w_ref[...], preferred_element_type=jnp.float32)
    m = jax.nn.relu(hsrc + e + b_ref[...])
    o_ref[...] = jnp.dot(m, nw_ref[...], preferred_element_type=jnp.float32)


def _edge1(c, ea, emb, w, b, nw):
    return pl.pallas_call(
        _edge1_body,
        grid=(EP // BE,),
        in_specs=[
            pl.BlockSpec((BE,), lambda i: (i,)),
            pl.BlockSpec((BE, 16), lambda i: (jnp.minimum(i, E // BE - 1), 0)),
            pl.BlockSpec((20, 128), lambda i: (0, 0)),
            pl.BlockSpec((16, 128), lambda i: (0, 0)),
            pl.BlockSpec((1, 128), lambda i: (0, 0)),
            pl.BlockSpec((128, 128), lambda i: (0, 0)),
        ],
        out_specs=pl.BlockSpec((BE, 128), lambda i: (i, 0)),
        out_shape=jax.ShapeDtypeStruct((EP, 128), jnp.float32),
    )(c, ea, emb, w, b, nw)


def _edge2_body(g_ref, ea_ref, w_ref, b_ref, nw_ref, o_ref):
    e = jnp.dot(ea_ref[...], w_ref[...], preferred_element_type=jnp.float32)
    m = jax.nn.relu(g_ref[...] + e + b_ref[...])
    o_ref[...] = jnp.dot(m, nw_ref[...], preferred_element_type=jnp.float32)


def _edge2(g, ea, w, b, nw):
    return pl.pallas_call(
        _edge2_body,
        grid=(EP // BE,),
        in_specs=[
            pl.BlockSpec((BE, 64), lambda i: (i, 0)),
            pl.BlockSpec((BE, 16), lambda i: (jnp.minimum(i, E // BE - 1), 0)),
            pl.BlockSpec((16, 64), lambda i: (0, 0)),
            pl.BlockSpec((1, 64), lambda i: (0, 0)),
            pl.BlockSpec((64, 128), lambda i: (0, 0)),
        ],
        out_specs=pl.BlockSpec((BE, 128), lambda i: (i, 0)),
        out_shape=jax.ShapeDtypeStruct((EP, 128), jnp.float32),
    )(g, ea, w, b, nw)


def _gram_body(a_ref, b_ref, o_ref):
    o_ref[...] = jax.lax.dot_general(
        a_ref[...], b_ref[...], (((1,), (1,)), ((), ())),
        preferred_element_type=jnp.float32)


def _gram(h):
    n = h.shape[0]
    grid = (pl.cdiv(n, BM), pl.cdiv(n, BN))
    return pl.pallas_call(
        _gram_body,
        grid=grid,
        in_specs=[
            pl.BlockSpec((BM, h.shape[1]), lambda i, j: (i, 0)),
            pl.BlockSpec((BN, h.shape[1]), lambda i, j: (j, 0)),
        ],
        out_specs=pl.BlockSpec((BM, BN), lambda i, j: (i, j)),
        out_shape=jax.ShapeDtypeStruct((n, n), jnp.float32),
    )(h, h)


def _pad128(w):
    return jnp.pad(w, ((0, 0), (0, 128 - w.shape[1])))


def kernel(x, edge_index, edge_attr, emb, lin_edge1_w, lin_edge1_b, nn1_w, nn1_b,
           lin_edge2_w, lin_edge2_b, nn2_w, nn2_b):
    src = jnp.concatenate([edge_index[0], jnp.zeros((EP - E,), jnp.int32)])
    dst = jnp.concatenate(
        [edge_index[1], jnp.full((EP - E,), N, jnp.int32)])
    src3 = src.reshape(NW, EP // (NW * CH), CH)
    dst3 = dst.reshape(NW, EP // (NW * SCH), SCH)
    hx = jnp.take(emb, x[:, 0], axis=0)

    c = _sc_class_gather(x[:, 0], src3)
    p1 = _edge1(c, edge_attr, emb, lin_edge1_w, lin_edge1_b.reshape(1, 128),
                _pad128(nn1_w))
    part = _sc_scatter_add(p1, dst3, jnp.zeros((NACC, 128), jnp.float32),
                           nbuf=2, look=1)
    aggr = (part[0] + part[1])[:N, :64]
    h1 = jax.nn.leaky_relu(aggr + (1.0 + EPS) * (hx @ nn1_w) + nn1_b,
                           negative_slope=0.01)

    g = _sc_gather(h1, src3)
    p2 = _edge2(g, edge_attr, lin_edge2_w, lin_edge2_b.reshape(1, 64),
                _pad128(nn2_w))
    part = _sc_scatter_add(p2, dst3, jnp.zeros((NACC, 32), jnp.float32),
                           nbuf=4, look=2)
    aggr = (part[0] + part[1])[:N]
    h2 = aggr + (1.0 + EPS) * (h1 @ nn2_w) + nn2_b

    return _gram(h2)


# speedup vs baseline: 1.5408x; 1.1139x over previous
import functools

import jax
import jax.numpy as jnp
from jax import lax
from jax.experimental import pallas as pl
from jax.experimental.pallas import tpu as pltpu
from jax.experimental.pallas import tpu_sc as plsc

EPS = 1e-09
N = 10000
E = 640000
NW = 32
CH = 64
EP = 655360
NACC = 10112
NBUF = 5
LOOK = 3
SCH = 64
BE = 5120
BM = 1024
BN = 1280


def _sc_class_gather(xflat, idx):
    ep = idx.shape[0]
    per_w = ep // NW
    mesh = plsc.VectorSubcoreMesh(core_axis_name="c", subcore_axis_name="s")

    @functools.partial(
        pl.kernel,
        out_type=jax.ShapeDtypeStruct((ep,), jnp.int32),
        mesh=mesh,
        compiler_params=pltpu.CompilerParams(use_tc_tiling_on_sc=False,
                                             needs_layout_passes=False),
        scratch_types=[
            pltpu.VMEM((N,), jnp.int32),
            pltpu.VMEM((per_w,), jnp.int32),
            pltpu.VMEM((per_w,), jnp.int32),
        ],
    )
    def k(x_hbm, idx_hbm, out_hbm, x_v, idx_v, c_v):
        wid = lax.axis_index("s") * 2 + lax.axis_index("c")
        pltpu.sync_copy(x_hbm, x_v)
        pltpu.sync_copy(idx_hbm.at[wid], idx_v)

        def grp(t, carry):
            s16 = idx_v[pl.ds(t * 16, 16)]
            c_v[pl.ds(t * 16, 16)] = plsc.load_gather(x_v, [s16])
            return carry

        lax.fori_loop(0, per_w // 16, grp, 0)
        pltpu.sync_copy(c_v, out_hbm.at[pl.ds(wid * per_w, per_w)])

    return k(xflat, idx.reshape(NW, per_w))


def _sc_gather(table, idx, ch, nbuf, look):
    ep = idx.shape[0]
    per_w = ep // NW
    nch = per_w // ch
    d = table.shape[1]
    mesh = plsc.VectorSubcoreMesh(core_axis_name="c", subcore_axis_name="s")

    @functools.partial(
        pl.kernel,
        out_type=jax.ShapeDtypeStruct((ep, d), jnp.float32),
        mesh=mesh,
        compiler_params=pltpu.CompilerParams(use_tc_tiling_on_sc=False),
        scratch_types=[
            pltpu.VMEM((per_w,), jnp.int32),
            pltpu.VMEM((nbuf, ch, d), jnp.float32),
            pltpu.SemaphoreType.DMA((nbuf,)),
            pltpu.SemaphoreType.DMA((nbuf,)),
        ],
    )
    def k(table_hbm, idx_hbm, out_hbm, idx_v, bufs_v, gsem, ssem):
        wid = lax.axis_index("s") * 2 + lax.axis_index("c")
        row0 = wid * nch
        pltpu.sync_copy(idx_hbm.at[pl.ds(wid * per_w, per_w)], idx_v)

        def idx_at(j):
            return idx_v.at[pl.ds(j * ch, ch)]

        for j in range(look):
            pltpu.async_copy(table_hbm.at[idx_at(j)], bufs_v.at[j], gsem.at[j])

        def group(g, carry):
            for b in range(nbuf):
                j = g * nbuf + b
                pltpu.make_async_copy(
                    table_hbm.at[idx_at(0)], bufs_v.at[b], gsem.at[b]).wait()
                pltpu.async_copy(
                    bufs_v.at[b], out_hbm.at[pl.ds((row0 + j) * ch, ch)],
                    ssem.at[b])
                jn = j + look
                bn = (b + look) % nbuf

                @pl.when(jn < nch)
                def _():
                    @pl.when(jn >= nbuf)
                    def _():
                        pltpu.make_async_copy(
                            bufs_v.at[bn], out_hbm.at[pl.ds(0, ch)],
                            ssem.at[bn]).wait()
                    pltpu.async_copy(
                        table_hbm.at[idx_at(jn)], bufs_v.at[bn], gsem.at[bn])
            return carry

        lax.fori_loop(0, nch // nbuf, group, 0)
        for b in range(nbuf):
            pltpu.make_async_copy(
                bufs_v.at[b], out_hbm.at[pl.ds(0, ch)], ssem.at[b]).wait()

    return k(table, idx)


def _sc_scatter_add(vals, idx, zeros, ch, nbuf, look):
    ep = idx.shape[0]
    per_w = ep // NW
    nch = per_w // ch
    d = zeros.shape[1]
    rpt = NACC // 16
    mesh = plsc.VectorSubcoreMesh(core_axis_name="c", subcore_axis_name="s")

    scratch = [
        pltpu.VMEM((per_w,), jnp.int32),
        pltpu.VMEM((nbuf, ch, 128), jnp.float32),
    ]
    if d < 128:
        scratch.append(pltpu.VMEM((nbuf, ch, d), jnp.float32))
    scratch += [
        pltpu.VMEM_SHARED((NACC, d), jnp.float32),
        pltpu.SemaphoreType.DMA((nbuf,)),
        pltpu.SemaphoreType.DMA((nbuf,)),
    ]

    @functools.partial(
        pl.kernel,
        out_type=jax.ShapeDtypeStruct((2, NACC, d), jnp.float32),
        mesh=mesh,
        compiler_params=pltpu.CompilerParams(use_tc_tiling_on_sc=False),
        scratch_types=scratch,
    )
    def k(vals_hbm, idx_hbm, zeros_hbm, out_hbm, idx_v, bufs_v, *rest):
        if d < 128:
            cbufs_v, acc_sh, gsem, ssem = rest
        else:
            acc_sh, gsem, ssem = rest
            cbufs_v = None
        cid = lax.axis_index("c")
        sid = lax.axis_index("s")
        wid = sid * 2 + cid
        row0 = wid * nch
        pltpu.sync_copy(zeros_hbm.at[pl.ds(sid * rpt, rpt)],
                        acc_sh.at[pl.ds(sid * rpt, rpt)])
        plsc.subcore_barrier()
        pltpu.sync_copy(idx_hbm.at[pl.ds(wid * per_w, per_w)], idx_v)

        def idx_at(j):
            return idx_v.at[pl.ds(j * ch, ch)]

        def src(b):
            return bufs_v.at[b] if d == 128 else cbufs_v.at[b]

        def repack(b):
            if d == 128:
                return
            unroll = 8

            def rows(r0, carry):
                for u in range(unroll):
                    r = r0 * unroll + u
                    for q in range(d // 16):
                        cbufs_v[b, r, pl.ds(q * 16, 16)] = (
                            bufs_v[b, r, pl.ds(q * 16, 16)])
                return carry

            lax.fori_loop(0, ch // unroll, rows, 0)

        for j in range(look):
            pltpu.async_copy(
                vals_hbm.at[pl.ds((row0 + j) * ch, ch)], bufs_v.at[j],
                gsem.at[j])

        def group(g, carry):
            for b in range(nbuf):
                j = g * nbuf + b
                pltpu.make_async_copy(
                    vals_hbm.at[pl.ds(0, ch)], bufs_v.at[b], gsem.at[b]).wait()
                repack(b)
                pltpu.async_copy(
                    src(b), acc_sh.at[idx_at(j)], ssem.at[b], add=True)
                jn = j + look
                bn = (b + look) % nbuf

                @pl.when(jn < nch)
                def _():
                    @pl.when(jn >= nbuf)
                    def _():
                        pltpu.make_async_copy(
                            src(bn), acc_sh.at[idx_at(0)],
                            ssem.at[bn]).wait()
                    pltpu.async_copy(
                        vals_hbm.at[pl.ds((row0 + jn) * ch, ch)], bufs_v.at[bn],
                        gsem.at[bn])
            return carry

        lax.fori_loop(0, nch // nbuf, group, 0)
        for b in range(nbuf):
            pltpu.make_async_copy(
                src(b), acc_sh.at[idx_at(0)], ssem.at[b]).wait()
        plsc.subcore_barrier()
        pltpu.sync_copy(acc_sh.at[pl.ds(sid * rpt, rpt)],
                        out_hbm.at[cid].at[pl.ds(sid * rpt, rpt)])

    return k(vals, idx, zeros)


def _edge1_body(c_ref, ea_ref, emb_ref, w_ref, b_ref, nw_ref, o_ref):
    c = c_ref[...]
    oht = (jax.lax.broadcasted_iota(jnp.int32, (20, BE), 0)
           == c[None, :]).astype(jnp.float32)
    hsrc = jax.lax.dot_general(
        oht, emb_ref[...], (((0,), (0,)), ((), ())),
        preferred_element_type=jnp.float32)
    e = jnp.dot(ea_ref[...], w_ref[...], preferred_element_type=jnp.float32)
    m = jax.nn.relu(hsrc + e + b_ref[...])
    o_ref[...] = jnp.dot(m, nw_ref[...], preferred_element_type=jnp.float32)


def _edge1(c, ea, emb, w, b, nw):
    return pl.pallas_call(
        _edge1_body,
        grid=(EP // BE,),
        in_specs=[
            pl.BlockSpec((BE,), lambda i: (i,)),
            pl.BlockSpec((BE, 16), lambda i: (jnp.minimum(i, E // BE - 1), 0)),
            pl.BlockSpec((20, 128), lambda i: (0, 0)),
            pl.BlockSpec((16, 128), lambda i: (0, 0)),
            pl.BlockSpec((1, 128), lambda i: (0, 0)),
            pl.BlockSpec((128, 128), lambda i: (0, 0)),
        ],
        out_specs=pl.BlockSpec((BE, 128), lambda i: (i, 0)),
        out_shape=jax.ShapeDtypeStruct((EP, 128), jnp.float32),
    )(c, ea, emb, w, b, nw)


def _edge2_body(g_ref, ea_ref, w_ref, b_ref, nw_ref, o_ref):
    e = jnp.dot(ea_ref[...], w_ref[...], preferred_element_type=jnp.float32)
    m = jax.nn.relu(g_ref[...] + e + b_ref[...])
    o_ref[...] = jnp.dot(m, nw_ref[...], preferred_element_type=jnp.float32)


def _edge2(g, ea, w, b, nw):
    return pl.pallas_call(
        _edge2_body,
        grid=(EP // BE,),
        in_specs=[
            pl.BlockSpec((BE, 64), lambda i: (i, 0)),
            pl.BlockSpec((BE, 16), lambda i: (jnp.minimum(i, E // BE - 1), 0)),
            pl.BlockSpec((16, 64), lambda i: (0, 0)),
            pl.BlockSpec((1, 64), lambda i: (0, 0)),
            pl.BlockSpec((64, 128), lambda i: (0, 0)),
        ],
        out_specs=pl.BlockSpec((BE, 128), lambda i: (i, 0)),
        out_shape=jax.ShapeDtypeStruct((EP, 128), jnp.float32),
    )(g, ea, w, b, nw)


def _gram_body(a_ref, b_ref, o_ref):
    o_ref[...] = jax.lax.dot_general(
        a_ref[...], b_ref[...], (((1,), (1,)), ((), ())),
        preferred_element_type=jnp.float32)


def _gram(h):
    n = h.shape[0]
    grid = (pl.cdiv(n, BM), pl.cdiv(n, BN))
    return pl.pallas_call(
        _gram_body,
        grid=grid,
        in_specs=[
            pl.BlockSpec((BM, h.shape[1]), lambda i, j: (i, 0)),
            pl.BlockSpec((BN, h.shape[1]), lambda i, j: (j, 0)),
        ],
        out_specs=pl.BlockSpec((BM, BN), lambda i, j: (i, j)),
        out_shape=jax.ShapeDtypeStruct((n, n), jnp.float32),
    )(h, h)


def _pad128(w):
    return jnp.pad(w, ((0, 0), (0, 128 - w.shape[1])))


def kernel(x, edge_index, edge_attr, emb, lin_edge1_w, lin_edge1_b, nn1_w, nn1_b,
           lin_edge2_w, lin_edge2_b, nn2_w, nn2_b):
    src = jnp.concatenate([edge_index[0], jnp.zeros((EP - E,), jnp.int32)])
    dst = jnp.concatenate(
        [edge_index[1], jnp.full((EP - E,), N, jnp.int32)])
    hx = jnp.take(emb, x[:, 0], axis=0)

    c = _sc_class_gather(x[:, 0], src)
    p1 = _edge1(c, edge_attr, emb, lin_edge1_w, lin_edge1_b.reshape(1, 128),
                _pad128(nn1_w))
    part = _sc_scatter_add(p1, dst, jnp.zeros((NACC, 128), jnp.float32),
                           ch=32, nbuf=4, look=3)
    aggr = (part[0] + part[1])[:N, :64]
    h1 = jax.nn.leaky_relu(aggr + (1.0 + EPS) * (hx @ nn1_w) + nn1_b,
                           negative_slope=0.01)

    g = _sc_gather(h1, src, ch=64, nbuf=8, look=6)
    p2 = _edge2(g, edge_attr, lin_edge2_w, lin_edge2_b.reshape(1, 64),
                _pad128(nn2_w))
    part = _sc_scatter_add(p2, dst, jnp.zeros((NACC, 32), jnp.float32),
                           ch=64, nbuf=8, look=6)
    aggr = (part[0] + part[1])[:N]
    h2 = aggr + (1.0 + EPS) * (h1 @ nn2_w) + nn2_b

    return _gram(h2)
